# Initial kernel scaffold; baseline (speedup 1.0000x reference)
import functools

import jax
import jax.numpy as jnp
from jax import lax
from jax.experimental import pallas as pl
from jax.experimental.pallas import tpu as pltpu
from jax.experimental.pallas import tpu_sc as plsc

_N = 10000
_E = 320000
_D = 128
_NC = 2
_NS = 16
_NW = _NC * _NS
_T = _E // _NW
_K = 80
_CH = _T // _K
_ZR = 125
_RPT = _N // _NS

_mesh = plsc.VectorSubcoreMesh(core_axis_name="c", subcore_axis_name="s")


@functools.partial(
    pl.kernel,
    out_type=jax.ShapeDtypeStruct((_NW, 2 * _N), jnp.float32),
    mesh=_mesh,
    scratch_types=[
        pltpu.VMEM((_T,), jnp.int32),
        pltpu.VMEM((_T,), jnp.int32),
        pltpu.VMEM((_T,), jnp.float32),
        pltpu.VMEM((2 * _N,), jnp.float32),
    ],
)
def _deg_kernel(src_hbm, dst_hbm, w_hbm, out_hbm, src_v, dst_v, w_v, acc):
    cid = lax.axis_index("c")
    sid = lax.axis_index("s")
    wid = cid * _NS + sid
    base = wid * _T
    pltpu.sync_copy(src_hbm.at[pl.ds(base, _T)], src_v)
    pltpu.sync_copy(dst_hbm.at[pl.ds(base, _T)], dst_v)
    pltpu.sync_copy(w_hbm.at[pl.ds(base, _T)], w_v)

    def zero_body(i, _):
        acc[pl.ds(i * 16, 16)] = jnp.zeros((16,), jnp.float32)
        return 0

    lax.fori_loop(0, 2 * _N // 16, zero_body, 0)

    def edge_body(i, _):
        sl = pl.ds(i * 16, 16)
        sv = src_v[sl]
        dv = dst_v[sl]
        wv = w_v[sl]
        plsc.addupdate_scatter(acc, [sv * 2], wv)
        plsc.addupdate_scatter(acc, [dv * 2 + 1], wv)
        return 0

    lax.fori_loop(0, _T // 16, edge_body, 0)
    pltpu.sync_copy(acc, out_hbm.at[wid])


@functools.partial(
    pl.kernel,
    out_type=jax.ShapeDtypeStruct((_NC, _N, _D), jnp.float32),
    mesh=_mesh,
    scratch_types=[
        pltpu.VMEM((_CH, _K), jnp.int32),
        pltpu.VMEM((_CH, _K), jnp.int32),
        pltpu.VMEM((_CH, _K), jnp.float32),
        pltpu.VMEM((_K, _D), jnp.float32),
        pltpu.VMEM((_ZR, _D), jnp.float32),
        pltpu.VMEM_SHARED((_N, _D), jnp.float32),
        pltpu.SemaphoreType.DMA,
    ],
)
def _layer_kernel(h_hbm, src_hbm, dst_hbm, w_hbm, out_hbm,
                  src_v, dst_v, w_v, rows, zbuf, acc, sem):
    cid = lax.axis_index("c")
    sid = lax.axis_index("s")
    wid = cid * _NS + sid
    pltpu.sync_copy(src_hbm.at[wid], src_v)
    pltpu.sync_copy(dst_hbm.at[wid], dst_v)
    pltpu.sync_copy(w_hbm.at[wid], w_v)

    def zb(i, _):
        for j in range(_D // 16):
            zbuf[i, pl.ds(j * 16, 16)] = jnp.zeros((16,), jnp.float32)
        return 0

    lax.fori_loop(0, _ZR, zb, 0)
    for r in range(_RPT // _ZR):
        pltpu.sync_copy(zbuf, acc.at[pl.ds(sid * _RPT + r * _ZR, _ZR)])
    plsc.subcore_barrier()

    def chunk(c, _):
        pltpu.async_copy(h_hbm.at[src_v.at[c]], rows, sem).wait()

        def scale(k, _):
            s = w_v[c, k]
            for j in range(_D // 16):
                sl = pl.ds(j * 16, 16)
                rows[k, sl] = rows[k, sl] * s
            return 0

        lax.fori_loop(0, _K, scale, 0)
        pltpu.sync_copy(rows, acc.at[dst_v.at[c]], add=True)
        return 0

    lax.fori_loop(0, _CH, chunk, 0)
    plsc.subcore_barrier()
    for r in range(_RPT // _ZR):
        off = sid * _RPT + r * _ZR
        pltpu.sync_copy(acc.at[pl.ds(off, _ZR)], out_hbm.at[cid, pl.ds(off, _ZR)])


_PREC = jax.lax.Precision.HIGHEST


def _tc_first_body(degp_ref, x_ref, w1_ref, norm_ref, h1_ref):
    deg = degp_ref[0]
    for i in range(1, _NW):
        deg = deg + degp_ref[i]
    norm = jax.lax.rsqrt(jnp.clip(deg, 1e-12, None))
    norm_ref[...] = norm
    h = jnp.dot(x_ref[...], w1_ref[...],
                preferred_element_type=jnp.float32, precision=_PREC)
    h1_ref[...] = h * norm[:, 0:1]


def _tc_mid_body(aggp_ref, norm_ref, b_ref, w_ref, h_ref):
    agg = aggp_ref[0] + aggp_ref[1]
    norm = norm_ref[...]
    out = jnp.maximum(agg * norm[:, 1:2] + b_ref[...], 0.0)
    h = jnp.dot(out, w_ref[...],
                preferred_element_type=jnp.float32, precision=_PREC)
    h_ref[...] = h * norm[:, 0:1]


def _tc_final_body(aggp_ref, norm_ref, b3_ref, x_ref, wres_ref, bres_ref,
                   wop_ref, bop_ref, out_ref):
    agg = aggp_ref[0] + aggp_ref[1]
    out3 = agg * norm_ref[...][:, 1:2] + b3_ref[...]
    res = jnp.dot(x_ref[...], wres_ref[...],
                  preferred_element_type=jnp.float32, precision=_PREC)
    res = res + bres_ref[...]
    y = jnp.maximum(out3 + res, 0.0)
    out_ref[...] = jnp.dot(y, wop_ref[...],
                           preferred_element_type=jnp.float32,
                           precision=_PREC) + bop_ref[...]


_tc_first = pl.pallas_call(
    _tc_first_body,
    out_shape=(
        jax.ShapeDtypeStruct((_N, 2), jnp.float32),
        jax.ShapeDtypeStruct((_N, _D), jnp.float32),
    ),
)

_tc_mid = pl.pallas_call(
    _tc_mid_body,
    out_shape=jax.ShapeDtypeStruct((_N, _D), jnp.float32),
)

_tc_final = pl.pallas_call(
    _tc_final_body,
    out_shape=jax.ShapeDtypeStruct((_N, 64), jnp.float32),
)


def kernel(x, edge_index, edge_weight, W1, b1, W2, b2, W3, b3,
           Wres, bres, Wop, bop):
    src = edge_index[0]
    dst = edge_index[1]
    src3 = src.reshape(_NW, _CH, _K)
    dst3 = dst.reshape(_NW, _CH, _K)
    w3 = edge_weight.reshape(_NW, _CH, _K)

    degp = _deg_kernel(src, dst, edge_weight)
    degp3 = degp.reshape(_NW, _N, 2)

    norm, h1 = _tc_first(degp3, x, W1)
    agg1 = _layer_kernel(h1, src3, dst3, w3)
    h2 = _tc_mid(agg1, norm, b1.reshape(1, _D), W2)
    agg2 = _layer_kernel(h2, src3, dst3, w3)
    h3 = _tc_mid(agg2, norm, b2.reshape(1, _D), W3)
    agg3 = _layer_kernel(h3, src3, dst3, w3)
    out = _tc_final(agg3, norm, b3.reshape(1, _D), x, Wres,
                    bres.reshape(1, _D), Wop, bop.reshape(1, 64))
    return out


# speedup vs baseline: 6.3341x; 6.3341x over previous
import functools

import jax
import jax.numpy as jnp
from jax import lax
from jax.experimental import pallas as pl
from jax.experimental.pallas import tpu as pltpu
from jax.experimental.pallas import tpu_sc as plsc

_N = 10000
_E = 320000
_D = 128
_C = 64
_NC = 2
_NS = 16
_NW = _NC * _NS
_T = _E // _NW
_K = 80
_CH = _T // _K
_NP = 10240
_RPT = _NP // _NS

_mesh = plsc.VectorSubcoreMesh(core_axis_name="c", subcore_axis_name="s")


@functools.partial(
    pl.kernel,
    out_type=jax.ShapeDtypeStruct((_NC, 2, _NS, _RPT, 16), jnp.float32),
    mesh=_mesh,
    compiler_params=pltpu.CompilerParams(use_tc_tiling_on_sc=False),
    scratch_types=[
        pltpu.VMEM((_CH, _K), jnp.int32),
        pltpu.VMEM((_CH, _K), jnp.int32),
        pltpu.VMEM((_CH, _K), jnp.float32),
        pltpu.VMEM((_K, 16), jnp.float32),
        pltpu.VMEM((_RPT, 16), jnp.float32),
        pltpu.VMEM_SHARED((_NP, 16), jnp.float32),
        pltpu.VMEM_SHARED((_NP, 16), jnp.float32),
    ],
)
def _deg_kernel(src_hbm, dst_hbm, w_hbm, out_hbm,
                src_v, dst_v, w_v, rows, zbuf, acc_o, acc_i):
    cid = lax.axis_index("c")
    sid = lax.axis_index("s")
    wid = cid * _NS + sid
    pltpu.sync_copy(src_hbm.at[wid], src_v)
    pltpu.sync_copy(dst_hbm.at[wid], dst_v)
    pltpu.sync_copy(w_hbm.at[wid], w_v)

    def zb(i, _):
        zbuf[i, pl.ds(0, 16)] = jnp.zeros((16,), jnp.float32)
        return 0

    lax.fori_loop(0, _RPT, zb, 0)
    pltpu.sync_copy(zbuf, acc_o.at[pl.ds(sid * _RPT, _RPT)])
    pltpu.sync_copy(zbuf, acc_i.at[pl.ds(sid * _RPT, _RPT)])
    plsc.subcore_barrier()

    def chunk(c, _):
        def splat(g, _):
            wv = w_v[c, pl.ds(g * 16, 16)]
            for l in range(16):
                rows[g * 16 + l, pl.ds(0, 16)] = jnp.full((16,), wv[l],
                                                          jnp.float32)
            return 0

        lax.fori_loop(0, _K // 16, splat, 0)
        pltpu.sync_copy(rows, acc_o.at[src_v.at[c]], add=True)
        pltpu.sync_copy(rows, acc_i.at[dst_v.at[c]], add=True)
        return 0

    lax.fori_loop(0, _CH, chunk, 0)
    plsc.subcore_barrier()
    sl = pl.ds(sid * _RPT, _RPT)
    pltpu.sync_copy(acc_o.at[sl], out_hbm.at[cid, 0, sid])
    pltpu.sync_copy(acc_i.at[sl], out_hbm.at[cid, 1, sid])


@functools.partial(
    pl.kernel,
    out_type=jax.ShapeDtypeStruct((_NC, _NS, _RPT, _D), jnp.float32),
    mesh=_mesh,
    compiler_params=pltpu.CompilerParams(use_tc_tiling_on_sc=False),
    scratch_types=[
        pltpu.VMEM((_CH, _K), jnp.int32),
        pltpu.VMEM((_CH, _K), jnp.int32),
        pltpu.VMEM((_CH, _K), jnp.float32),
        pltpu.VMEM((_K, _D), jnp.float32),
        pltpu.VMEM_SHARED((_NP, _D), jnp.float32),
        pltpu.SemaphoreType.DMA,
    ],
)
def _layer_kernel(h_hbm, src_hbm, dst_hbm, w_hbm, out_hbm,
                  src_v, dst_v, w_v, rows, acc, sem):
    cid = lax.axis_index("c")
    sid = lax.axis_index("s")
    wid = cid * _NS + sid
    pltpu.sync_copy(src_hbm.at[wid], src_v)
    pltpu.sync_copy(dst_hbm.at[wid], dst_v)
    pltpu.sync_copy(w_hbm.at[wid], w_v)

    def zb(i, _):
        for j in range(_D // 16):
            rows[i, pl.ds(j * 16, 16)] = jnp.zeros((16,), jnp.float32)
        return 0

    lax.fori_loop(0, _K, zb, 0)
    for r in range(_RPT // _K):
        pltpu.sync_copy(rows, acc.at[pl.ds(sid * _RPT + r * _K, _K)])
    plsc.subcore_barrier()

    def chunk(c, _):
        pltpu.async_copy(h_hbm.at[src_v.at[c]], rows, sem).wait()

        def scale(g, _):
            wv = w_v[c, pl.ds(g * 16, 16)]
            for l in range(16):
                s = wv[l]
                k = g * 16 + l
                for j in range(_D // 16):
                    sl = pl.ds(j * 16, 16)
                    rows[k, sl] = rows[k, sl] * s
            return 0

        lax.fori_loop(0, _K // 16, scale, 0)
        pltpu.sync_copy(rows, acc.at[dst_v.at[c]], add=True)
        return 0

    lax.fori_loop(0, _CH, chunk, 0)
    plsc.subcore_barrier()
    pltpu.sync_copy(acc.at[pl.ds(sid * _RPT, _RPT)], out_hbm.at[cid, sid])


_PREC = jax.lax.Precision.HIGHEST


def _tc_first_body(degp_ref, x_ref, w1_ref, norm_ref, h1_ref):
    deg_out = degp_ref[0, 0, 0:_N, 0:1] + degp_ref[1, 0, 0:_N, 0:1]
    deg_in = degp_ref[0, 1, 0:_N, 0:1] + degp_ref[1, 1, 0:_N, 0:1]
    deg = jnp.concatenate([deg_out, deg_in], axis=1)
    norm = jax.lax.rsqrt(jnp.clip(deg, 1e-12, None))
    norm_ref[...] = norm
    h = jnp.dot(x_ref[...], w1_ref[...],
                preferred_element_type=jnp.float32, precision=_PREC)
    h1_ref[...] = h * norm[:, 0:1]


def _tc_mid_body(aggp_ref, norm_ref, b_ref, w_ref, h_ref):
    agg = aggp_ref[0, 0:_N] + aggp_ref[1, 0:_N]
    norm = norm_ref[...]
    out = jnp.maximum(agg * norm[:, 1:2] + b_ref[...], 0.0)
    h = jnp.dot(out, w_ref[...],
                preferred_element_type=jnp.float32, precision=_PREC)
    h_ref[...] = h * norm[:, 0:1]


def _tc_final_body(aggp_ref, norm_ref, b3_ref, x_ref, wres_ref, bres_ref,
                   wop_ref, bop_ref, out_ref):
    agg = aggp_ref[0, 0:_N] + aggp_ref[1, 0:_N]
    out3 = agg * norm_ref[...][:, 1:2] + b3_ref[...]
    res = jnp.dot(x_ref[...], wres_ref[...],
                  preferred_element_type=jnp.float32, precision=_PREC)
    res = res + bres_ref[...]
    y = jnp.maximum(out3 + res, 0.0)
    out_ref[...] = jnp.dot(y, wop_ref[...],
                           preferred_element_type=jnp.float32,
                           precision=_PREC) + bop_ref[...]


_TC_PARAMS = pltpu.CompilerParams(vmem_limit_bytes=100 * 1024 * 1024)

_tc_first = pl.pallas_call(
    _tc_first_body,
    compiler_params=_TC_PARAMS,
    out_shape=(
        jax.ShapeDtypeStruct((_N, 2), jnp.float32),
        jax.ShapeDtypeStruct((_N, _D), jnp.float32),
    ),
)

_tc_mid = pl.pallas_call(
    _tc_mid_body,
    compiler_params=_TC_PARAMS,
    out_shape=jax.ShapeDtypeStruct((_N, _D), jnp.float32),
)

_tc_final = pl.pallas_call(
    _tc_final_body,
    compiler_params=_TC_PARAMS,
    out_shape=jax.ShapeDtypeStruct((_N, _C), jnp.float32),
)


def kernel(x, edge_index, edge_weight, W1, b1, W2, b2, W3, b3,
           Wres, bres, Wop, bop):
    src3 = edge_index[0].reshape(_NW, _CH, _K)
    dst3 = edge_index[1].reshape(_NW, _CH, _K)
    w3 = edge_weight.reshape(_NW, _CH, _K)

    degp = _deg_kernel(src3, dst3, w3)
    degp = degp.reshape(_NC, 2, _NP, 16)

    norm, h1 = _tc_first(degp, x, W1)
    agg1 = _layer_kernel(h1, src3, dst3, w3).reshape(_NC, _NP, _D)
    h2 = _tc_mid(agg1, norm, b1.reshape(1, _D), W2)
    agg2 = _layer_kernel(h2, src3, dst3, w3).reshape(_NC, _NP, _D)
    h3 = _tc_mid(agg2, norm, b2.reshape(1, _D), W3)
    agg3 = _layer_kernel(h3, src3, dst3, w3).reshape(_NC, _NP, _D)
    out = _tc_final(agg3, norm, b3.reshape(1, _D), x, Wres,
                    bres.reshape(1, _D), Wop, bop.reshape(1, _C))
    return out


# speedup vs baseline: 7.8403x; 1.2378x over previous
import functools

import jax
import jax.numpy as jnp
from jax import lax
from jax.experimental import pallas as pl
from jax.experimental.pallas import tpu as pltpu
from jax.experimental.pallas import tpu_sc as plsc

_N = 10000
_E = 320000
_D = 128
_C = 64
_NC = 2
_NS = 16
_NW = _NC * _NS
_T = _E // _NW
_K = 40
_CH = _T // _K
_NP = 10240
_RPT = _NP // _NS

_WGROUPS = ((0, 0), (16, 0), (24, 8))

_mesh = plsc.VectorSubcoreMesh(core_axis_name="c", subcore_axis_name="s")


@functools.partial(
    pl.kernel,
    out_type=jax.ShapeDtypeStruct((_NC, 2, _NS, _RPT, 16), jnp.float32),
    mesh=_mesh,
    compiler_params=pltpu.CompilerParams(use_tc_tiling_on_sc=False),
    scratch_types=[
        pltpu.VMEM((_CH, _K), jnp.int32),
        pltpu.VMEM((_CH, _K), jnp.int32),
        pltpu.VMEM((_CH, _K), jnp.float32),
        pltpu.VMEM((_K, 16), jnp.float32),
        pltpu.VMEM((_RPT, 16), jnp.float32),
        pltpu.VMEM_SHARED((_NP, 16), jnp.float32),
        pltpu.VMEM_SHARED((_NP, 16), jnp.float32),
    ],
)
def _deg_kernel(src_hbm, dst_hbm, w_hbm, out_hbm,
                src_v, dst_v, w_v, rows, zbuf, acc_o, acc_i):
    cid = lax.axis_index("c")
    sid = lax.axis_index("s")
    wid = cid * _NS + sid
    pltpu.sync_copy(src_hbm.at[wid], src_v)
    pltpu.sync_copy(dst_hbm.at[wid], dst_v)
    pltpu.sync_copy(w_hbm.at[wid], w_v)

    def zb(i, _):
        zbuf[i, pl.ds(0, 16)] = jnp.zeros((16,), jnp.float32)
        return 0

    lax.fori_loop(0, _RPT, zb, 0)
    pltpu.sync_copy(zbuf, acc_o.at[pl.ds(sid * _RPT, _RPT)])
    pltpu.sync_copy(zbuf, acc_i.at[pl.ds(sid * _RPT, _RPT)])
    plsc.subcore_barrier()

    def chunk(c, _):
        for off, ls in _WGROUPS:
            wv = w_v[c, pl.ds(off, 16)]
            for l in range(ls, 16):
                rows[off + l, pl.ds(0, 16)] = jnp.full((16,), wv[l],
                                                       jnp.float32)
        pltpu.sync_copy(rows, acc_o.at[src_v.at[c]], add=True)
        pltpu.sync_copy(rows, acc_i.at[dst_v.at[c]], add=True)
        return 0

    lax.fori_loop(0, _CH, chunk, 0)
    plsc.subcore_barrier()
    sl = pl.ds(sid * _RPT, _RPT)
    pltpu.sync_copy(acc_o.at[sl], out_hbm.at[cid, 0, sid])
    pltpu.sync_copy(acc_i.at[sl], out_hbm.at[cid, 1, sid])


@functools.partial(
    pl.kernel,
    out_type=jax.ShapeDtypeStruct((_NC, _NS, _RPT, _D), jnp.float32),
    mesh=_mesh,
    compiler_params=pltpu.CompilerParams(use_tc_tiling_on_sc=False),
    scratch_types=[
        pltpu.VMEM((_CH, _K), jnp.int32),
        pltpu.VMEM((_CH, _K), jnp.int32),
        pltpu.VMEM((_CH, _K), jnp.float32),
        pltpu.VMEM((_K, _D), jnp.float32),
        pltpu.VMEM((_K, _D), jnp.float32),
        pltpu.VMEM_SHARED((_NP, _D), jnp.float32),
        pltpu.SemaphoreType.DMA,
        pltpu.SemaphoreType.DMA,
        pltpu.SemaphoreType.DMA,
        pltpu.SemaphoreType.DMA,
    ],
)
def _layer_kernel(h_hbm, src_hbm, dst_hbm, w_hbm, out_hbm,
                  src_v, dst_v, w_v, rows0, rows1, acc,
                  gs0, gs1, ss0, ss1):
    cid = lax.axis_index("c")
    sid = lax.axis_index("s")
    wid = cid * _NS + sid
    pltpu.sync_copy(src_hbm.at[wid], src_v)
    pltpu.sync_copy(dst_hbm.at[wid], dst_v)
    pltpu.sync_copy(w_hbm.at[wid], w_v)

    def zb(i, _):
        for j in range(_D // 16):
            rows0[i, pl.ds(j * 16, 16)] = jnp.zeros((16,), jnp.float32)
        return 0

    lax.fori_loop(0, _K, zb, 0)
    for r in range(_RPT // _K):
        pltpu.sync_copy(rows0, acc.at[pl.ds(sid * _RPT + r * _K, _K)])
    plsc.subcore_barrier()

    def scale(c, rows):
        for off, ls in _WGROUPS:
            wv = w_v[c, pl.ds(off, 16)]
            for l in range(ls, 16):
                k = off + l
                s = wv[l]
                for j in range(_D // 16):
                    sl = pl.ds(j * 16, 16)
                    rows[k, sl] = rows[k, sl] * s

    def gather(c, rows, sem):
        pltpu.async_copy(h_hbm.at[src_v.at[c]], rows, sem)

    def maybe_gather(c, rows, sem):
        @pl.when(c < _CH)
        def _():
            pltpu.async_copy(h_hbm.at[src_v.at[c]], rows, sem)

    gather(0, rows0, gs0)
    gather(1, rows1, gs1)

    def pair(cc, _):
        c0 = cc * 2
        c1 = c0 + 1
        pltpu.make_async_copy(h_hbm.at[src_v.at[c0]], rows0, gs0).wait()
        scale(c0, rows0)
        pltpu.async_copy(rows0, acc.at[dst_v.at[c0]], ss0, add=True)
        pltpu.make_async_copy(h_hbm.at[src_v.at[c1]], rows1, gs1).wait()
        scale(c1, rows1)
        pltpu.async_copy(rows1, acc.at[dst_v.at[c1]], ss1, add=True)
        pltpu.make_async_copy(rows0, acc.at[dst_v.at[c0]], ss0).wait()
        maybe_gather(c0 + 2, rows0, gs0)
        pltpu.make_async_copy(rows1, acc.at[dst_v.at[c1]], ss1).wait()
        maybe_gather(c1 + 2, rows1, gs1)
        return 0

    lax.fori_loop(0, _CH // 2, pair, 0)
    plsc.subcore_barrier()
    pltpu.sync_copy(acc.at[pl.ds(sid * _RPT, _RPT)], out_hbm.at[cid, sid])


_PREC = jax.lax.Precision.HIGHEST


def _tc_first_body(degp_ref, x_ref, w1_ref, norm_ref, h1_ref):
    deg_out = degp_ref[0, 0, 0:_N, 0:1] + degp_ref[1, 0, 0:_N, 0:1]
    deg_in = degp_ref[0, 1, 0:_N, 0:1] + degp_ref[1, 1, 0:_N, 0:1]
    deg = jnp.concatenate([deg_out, deg_in], axis=1)
    norm = jax.lax.rsqrt(jnp.clip(deg, 1e-12, None))
    norm_ref[...] = norm
    h = jnp.dot(x_ref[...], w1_ref[...],
                preferred_element_type=jnp.float32, precision=_PREC)
    h1_ref[...] = h * norm[:, 0:1]


def _tc_mid_body(aggp_ref, norm_ref, b_ref, w_ref, h_ref):
    agg = aggp_ref[0, 0:_N] + aggp_ref[1, 0:_N]
    norm = norm_ref[...]
    out = jnp.maximum(agg * norm[:, 1:2] + b_ref[...], 0.0)
    h = jnp.dot(out, w_ref[...],
                preferred_element_type=jnp.float32, precision=_PREC)
    h_ref[...] = h * norm[:, 0:1]


def _tc_final_body(aggp_ref, norm_ref, b3_ref, x_ref, wres_ref, bres_ref,
                   wop_ref, bop_ref, out_ref):
    agg = aggp_ref[0, 0:_N] + aggp_ref[1, 0:_N]
    out3 = agg * norm_ref[...][:, 1:2] + b3_ref[...]
    res = jnp.dot(x_ref[...], wres_ref[...],
                  preferred_element_type=jnp.float32, precision=_PREC)
    res = res + bres_ref[...]
    y = jnp.maximum(out3 + res, 0.0)
    out_ref[...] = jnp.dot(y, wop_ref[...],
                           preferred_element_type=jnp.float32,
                           precision=_PREC) + bop_ref[...]


_TC_PARAMS = pltpu.CompilerParams(vmem_limit_bytes=100 * 1024 * 1024)

_tc_first = pl.pallas_call(
    _tc_first_body,
    compiler_params=_TC_PARAMS,
    out_shape=(
        jax.ShapeDtypeStruct((_N, 2), jnp.float32),
        jax.ShapeDtypeStruct((_N, _D), jnp.float32),
    ),
)

_tc_mid = pl.pallas_call(
    _tc_mid_body,
    compiler_params=_TC_PARAMS,
    out_shape=jax.ShapeDtypeStruct((_N, _D), jnp.float32),
)

_tc_final = pl.pallas_call(
    _tc_final_body,
    compiler_params=_TC_PARAMS,
    out_shape=jax.ShapeDtypeStruct((_N, _C), jnp.float32),
)


def kernel(x, edge_index, edge_weight, W1, b1, W2, b2, W3, b3,
           Wres, bres, Wop, bop):
    src3 = edge_index[0].reshape(_NW, _CH, _K)
    dst3 = edge_index[1].reshape(_NW, _CH, _K)
    w3 = edge_weight.reshape(_NW, _CH, _K)

    degp = _deg_kernel(src3, dst3, w3)
    degp = degp.reshape(_NC, 2, _NP, 16)

    norm, h1 = _tc_first(degp, x, W1)
    agg1 = _layer_kernel(h1, src3, dst3, w3).reshape(_NC, _NP, _D)
    h2 = _tc_mid(agg1, norm, b1.reshape(1, _D), W2)
    agg2 = _layer_kernel(h2, src3, dst3, w3).reshape(_NC, _NP, _D)
    h3 = _tc_mid(agg2, norm, b2.reshape(1, _D), W3)
    agg3 = _layer_kernel(h3, src3, dst3, w3).reshape(_NC, _NP, _D)
    out = _tc_final(agg3, norm, b3.reshape(1, _D), x, Wres,
                    bres.reshape(1, _D), Wop, bop.reshape(1, _C))
    return out


# speedup vs baseline: 8.5140x; 1.0859x over previous
import functools

import jax
import jax.numpy as jnp
import numpy as np
from jax import lax
from jax.experimental import pallas as pl
from jax.experimental.pallas import tpu as pltpu
from jax.experimental.pallas import tpu_sc as plsc

_N = 10000
_E = 320000
_D = 128
_C = 64
_NC = 2
_NS = 16
_NW = _NC * _NS
_T = _E // _NW
_K = 40
_CH = _T // _K
_KD = 80
_CHD = _T // _KD
_NP = 10240
_RPT = _NP // _NS

_WGROUPS = ((0, 0), (16, 0), (24, 8))

_COLMAP = np.empty((_D,), np.int32)
for _g in range(_D // 32):
    for _i in range(16):
        _COLMAP[32 * _g + 2 * _i] = 32 * _g + _i
        _COLMAP[32 * _g + 2 * _i + 1] = 32 * _g + 16 + _i

_mesh = plsc.VectorSubcoreMesh(core_axis_name="c", subcore_axis_name="s")


@functools.partial(
    pl.kernel,
    out_type=jax.ShapeDtypeStruct((_NC, 2, _NS, _RPT, 16), jnp.float32),
    mesh=_mesh,
    compiler_params=pltpu.CompilerParams(use_tc_tiling_on_sc=False, needs_layout_passes=False),
    scratch_types=[
        pltpu.VMEM((_CHD, _KD), jnp.int32),
        pltpu.VMEM((_CHD, _KD), jnp.int32),
        pltpu.VMEM((_CHD, _KD), jnp.float32),
        pltpu.VMEM((_KD, 16), jnp.float32),
        pltpu.VMEM((_RPT, 16), jnp.float32),
        pltpu.VMEM_SHARED((_NP, 16), jnp.float32),
        pltpu.VMEM_SHARED((_NP, 16), jnp.float32),
    ],
)
def _deg_kernel(src_hbm, dst_hbm, w_hbm, out_hbm,
                src_v, dst_v, w_v, rows, zbuf, acc_o, acc_i):
    cid = lax.axis_index("c")
    sid = lax.axis_index("s")
    wid = cid * _NS + sid
    pltpu.sync_copy(src_hbm.at[wid], src_v)
    pltpu.sync_copy(dst_hbm.at[wid], dst_v)
    pltpu.sync_copy(w_hbm.at[wid], w_v)

    def zb(i, _):
        zbuf[i, pl.ds(0, 16)] = jnp.zeros((16,), jnp.float32)
        return 0

    lax.fori_loop(0, _RPT, zb, 0)
    pltpu.sync_copy(zbuf, acc_o.at[pl.ds(sid * _RPT, _RPT)])
    pltpu.sync_copy(zbuf, acc_i.at[pl.ds(sid * _RPT, _RPT)])
    plsc.subcore_barrier()

    def chunk(c, _):
        def splat(g, _):
            wv = w_v[c, pl.ds(g * 16, 16)]
            for l in range(16):
                rows[g * 16 + l, pl.ds(0, 16)] = jnp.full((16,), wv[l],
                                                          jnp.float32)
            return 0

        lax.fori_loop(0, _KD // 16, splat, 0)
        pltpu.sync_copy(rows, acc_o.at[src_v.at[c]], add=True)
        pltpu.sync_copy(rows, acc_i.at[dst_v.at[c]], add=True)
        return 0

    lax.fori_loop(0, _CHD, chunk, 0)
    plsc.subcore_barrier()
    sl = pl.ds(sid * _RPT, _RPT)
    pltpu.sync_copy(acc_o.at[sl], out_hbm.at[cid, 0, sid])
    pltpu.sync_copy(acc_i.at[sl], out_hbm.at[cid, 1, sid])


@functools.partial(
    pl.kernel,
    out_type=jax.ShapeDtypeStruct((_NC, _NS, _RPT, _D), jnp.float32),
    mesh=_mesh,
    compiler_params=pltpu.CompilerParams(use_tc_tiling_on_sc=False, needs_layout_passes=False),
    scratch_types=[
        pltpu.VMEM((_CH, _K), jnp.int32),
        pltpu.VMEM((_CH, _K), jnp.int32),
        pltpu.VMEM((_CH, _K), jnp.float32),
        pltpu.VMEM((_K, _D), jnp.bfloat16),
        pltpu.VMEM((_K, _D), jnp.bfloat16),
        pltpu.VMEM((_K, _D), jnp.float32),
        pltpu.VMEM((_K, _D), jnp.float32),
        pltpu.VMEM_SHARED((_NP, _D), jnp.float32),
        pltpu.SemaphoreType.DMA,
        pltpu.SemaphoreType.DMA,
        pltpu.SemaphoreType.DMA,
        pltpu.SemaphoreType.DMA,
    ],
)
def _layer_kernel(h_hbm, src_hbm, dst_hbm, w_hbm, out_hbm,
                  src_v, dst_v, w_v, brows0, brows1, rows0, rows1, acc,
                  gs0, gs1, ss0, ss1):
    cid = lax.axis_index("c")
    sid = lax.axis_index("s")
    wid = cid * _NS + sid
    pltpu.sync_copy(src_hbm.at[wid], src_v)
    pltpu.sync_copy(dst_hbm.at[wid], dst_v)
    pltpu.sync_copy(w_hbm.at[wid], w_v)

    def zb(i, _):
        for j in range(_D // 16):
            rows0[i, pl.ds(j * 16, 16)] = jnp.zeros((16,), jnp.float32)
        return 0

    lax.fori_loop(0, _K, zb, 0)
    for r in range(_RPT // _K):
        pltpu.sync_copy(rows0, acc.at[pl.ds(sid * _RPT + r * _K, _K)])
    plsc.subcore_barrier()

    def scale(c, brows, rows):
        for off, ls in _WGROUPS:
            wv = w_v[c, pl.ds(off, 16)]
            for l in range(ls, 16):
                k = off + l
                s = wv[l]
                for g in range(_D // 32):
                    v = brows[k, pl.ds(g * 32, 32)]
                    a, b = plsc.unpack(v, format=plsc.PackFormat.INTERLEAVED)
                    rows[k, pl.ds(g * 32, 16)] = a * s
                    rows[k, pl.ds(g * 32 + 16, 16)] = b * s

    def maybe_gather(c, brows, sem):
        @pl.when(c < _CH)
        def _():
            pltpu.async_copy(h_hbm.at[src_v.at[c]], brows, sem)

    pltpu.async_copy(h_hbm.at[src_v.at[0]], brows0, gs0)
    pltpu.async_copy(h_hbm.at[src_v.at[1]], brows1, gs1)

    def pair(cc, _):
        c0 = cc * 2
        c1 = c0 + 1
        pltpu.make_async_copy(h_hbm.at[src_v.at[c0]], brows0, gs0).wait()
        scale(c0, brows0, rows0)
        pltpu.async_copy(rows0, acc.at[dst_v.at[c0]], ss0, add=True)
        pltpu.make_async_copy(h_hbm.at[src_v.at[c1]], brows1, gs1).wait()
        scale(c1, brows1, rows1)
        pltpu.async_copy(rows1, acc.at[dst_v.at[c1]], ss1, add=True)
        pltpu.make_async_copy(rows0, acc.at[dst_v.at[c0]], ss0).wait()
        maybe_gather(c0 + 2, brows0, gs0)
        pltpu.make_async_copy(rows1, acc.at[dst_v.at[c1]], ss1).wait()
        maybe_gather(c1 + 2, brows1, gs1)
        return 0

    lax.fori_loop(0, _CH // 2, pair, 0)
    plsc.subcore_barrier()
    pltpu.sync_copy(acc.at[pl.ds(sid * _RPT, _RPT)], out_hbm.at[cid, sid])


_PREC = jax.lax.Precision.HIGHEST


def _tc_first_body(degp_ref, x_ref, w1_ref, norm_ref, h1_ref):
    deg_out = degp_ref[0, 0, 0:_N, 0:1] + degp_ref[1, 0, 0:_N, 0:1]
    deg_in = degp_ref[0, 1, 0:_N, 0:1] + degp_ref[1, 1, 0:_N, 0:1]
    deg = jnp.concatenate([deg_out, deg_in], axis=1)
    norm = jax.lax.rsqrt(jnp.clip(deg, 1e-12, None))
    norm_ref[...] = norm
    h = jnp.dot(x_ref[...], w1_ref[...],
                preferred_element_type=jnp.float32, precision=_PREC)
    h1_ref[...] = (h * norm[:, 0:1]).astype(jnp.bfloat16)


def _tc_mid_body(aggp_ref, norm_ref, b_ref, w_ref, h_ref):
    agg = aggp_ref[0, 0:_N] + aggp_ref[1, 0:_N]
    norm = norm_ref[...]
    out = jnp.maximum(agg * norm[:, 1:2] + b_ref[...], 0.0)
    h = jnp.dot(out, w_ref[...],
                preferred_element_type=jnp.float32, precision=_PREC)
    h_ref[...] = (h * norm[:, 0:1]).astype(jnp.bfloat16)


def _tc_final_body(aggp_ref, norm_ref, b3_ref, x_ref, wres_ref, bres_ref,
                   wop_ref, bop_ref, out_ref):
    agg = aggp_ref[0, 0:_N] + aggp_ref[1, 0:_N]
    out3 = agg * norm_ref[...][:, 1:2] + b3_ref[...]
    res = jnp.dot(x_ref[...], wres_ref[...],
                  preferred_element_type=jnp.float32, precision=_PREC)
    res = res + bres_ref[...]
    y = jnp.maximum(out3 + res, 0.0)
    out_ref[...] = jnp.dot(y, wop_ref[...],
                           preferred_element_type=jnp.float32,
                           precision=_PREC) + bop_ref[...]


_TC_PARAMS = pltpu.CompilerParams(vmem_limit_bytes=100 * 1024 * 1024)

_tc_first = pl.pallas_call(
    _tc_first_body,
    compiler_params=_TC_PARAMS,
    out_shape=(
        jax.ShapeDtypeStruct((_N, 2), jnp.float32),
        jax.ShapeDtypeStruct((_N, _D), jnp.bfloat16),
    ),
)

_tc_mid = pl.pallas_call(
    _tc_mid_body,
    compiler_params=_TC_PARAMS,
    out_shape=jax.ShapeDtypeStruct((_N, _D), jnp.bfloat16),
)

_tc_final = pl.pallas_call(
    _tc_final_body,
    compiler_params=_TC_PARAMS,
    out_shape=jax.ShapeDtypeStruct((_N, _C), jnp.float32),
)


def kernel(x, edge_index, edge_weight, W1, b1, W2, b2, W3, b3,
           Wres, bres, Wop, bop):
    src3 = edge_index[0].reshape(_NW, _CH, _K)
    dst3 = edge_index[1].reshape(_NW, _CH, _K)
    w3 = edge_weight.reshape(_NW, _CH, _K)
    src3d = edge_index[0].reshape(_NW, _CHD, _KD)
    dst3d = edge_index[1].reshape(_NW, _CHD, _KD)
    w3d = edge_weight.reshape(_NW, _CHD, _KD)
    colmap = jnp.asarray(_COLMAP)
    w1p = W1[:, colmap]
    w2p = W2[:, colmap]
    w3p = W3[:, colmap]

    degp = _deg_kernel(src3d, dst3d, w3d)
    degp = degp.reshape(_NC, 2, _NP, 16)

    norm, h1 = _tc_first(degp, x, w1p)
    agg1 = _layer_kernel(h1, src3, dst3, w3).reshape(_NC, _NP, _D)
    h2 = _tc_mid(agg1, norm, b1.reshape(1, _D), w2p)
    agg2 = _layer_kernel(h2, src3, dst3, w3).reshape(_NC, _NP, _D)
    h3 = _tc_mid(agg2, norm, b2.reshape(1, _D), w3p)
    agg3 = _layer_kernel(h3, src3, dst3, w3).reshape(_NC, _NP, _D)
    out = _tc_final(agg3, norm, b3.reshape(1, _D), x, Wres,
                    bres.reshape(1, _D), Wop, bop.reshape(1, _C))
    return out


# speedup vs baseline: 8.8312x; 1.0373x over previous
import functools

import jax
import jax.numpy as jnp
import numpy as np
from jax import lax
from jax.experimental import pallas as pl
from jax.experimental.pallas import tpu as pltpu
from jax.experimental.pallas import tpu_sc as plsc

_N = 10000
_E = 320000
_D = 128
_C = 64
_NC = 2
_NS = 16
_NW = _NC * _NS
_T = _E // _NW
_K = 40
_CH = _T // _K
_KD = 80
_CHD = _T // _KD
_NP = 10240
_RPT = _NP // _NS

_WGROUPS = ((0, 0), (16, 0), (24, 8))

_COLMAP = np.empty((_D,), np.int32)
for _g in range(_D // 32):
    for _i in range(16):
        _COLMAP[32 * _g + 2 * _i] = 32 * _g + _i
        _COLMAP[32 * _g + 2 * _i + 1] = 32 * _g + 16 + _i

_mesh = plsc.VectorSubcoreMesh(core_axis_name="c", subcore_axis_name="s")


_SEG = 2 * _NP // _NS


@functools.partial(
    pl.kernel,
    out_type=jax.ShapeDtypeStruct((_NC, 2 * _NP), jnp.float32),
    mesh=_mesh,
    compiler_params=pltpu.CompilerParams(use_tc_tiling_on_sc=False, needs_layout_passes=False),
    scratch_types=[
        pltpu.VMEM((_T,), jnp.int32),
        pltpu.VMEM((_T,), jnp.int32),
        pltpu.VMEM((_T,), jnp.float32),
        pltpu.VMEM((2 * _NP,), jnp.float32),
        pltpu.VMEM((_SEG,), jnp.float32),
        pltpu.VMEM_SHARED((_NS, 2 * _NP), jnp.float32),
    ],
)
def _deg_kernel(src_hbm, dst_hbm, w_hbm, out_hbm,
                src_v, dst_v, w_v, acc, seg, stage):
    cid = lax.axis_index("c")
    sid = lax.axis_index("s")
    wid = cid * _NS + sid
    base = wid * _T
    pltpu.sync_copy(src_hbm.at[pl.ds(base, _T)], src_v)
    pltpu.sync_copy(dst_hbm.at[pl.ds(base, _T)], dst_v)
    pltpu.sync_copy(w_hbm.at[pl.ds(base, _T)], w_v)

    def zero(i, _):
        acc[pl.ds(i * 16, 16)] = jnp.zeros((16,), jnp.float32)
        return 0

    lax.fori_loop(0, 2 * _NP // 16, zero, 0)

    def edge(i, _):
        sl = pl.ds(i * 16, 16)
        sv = src_v[sl]
        dv = dst_v[sl]
        wv = w_v[sl]
        plsc.addupdate_scatter(acc, [sv * 2], wv)
        plsc.addupdate_scatter(acc, [dv * 2 + 1], wv)
        return 0

    lax.fori_loop(0, _T // 16, edge, 0)

    pltpu.sync_copy(acc, stage.at[sid])
    plsc.subcore_barrier()
    off = sid * _SEG

    def peer(t, _):
        tt = lax.rem(sid + t, _NS)
        pltpu.sync_copy(stage.at[tt, pl.ds(off, _SEG)], seg)

        def add16(i, _):
            sl = pl.ds(off + i * 16, 16)
            acc[sl] = acc[sl] + seg[pl.ds(i * 16, 16)]
            return 0

        lax.fori_loop(0, _SEG // 16, add16, 0)
        return 0

    lax.fori_loop(1, _NS, peer, 0)
    pltpu.sync_copy(acc.at[pl.ds(off, _SEG)], out_hbm.at[cid, pl.ds(off, _SEG)])


@functools.partial(
    pl.kernel,
    out_type=jax.ShapeDtypeStruct((_NC, _NS, _RPT, _D), jnp.float32),
    mesh=_mesh,
    compiler_params=pltpu.CompilerParams(use_tc_tiling_on_sc=False, needs_layout_passes=False),
    scratch_types=[
        pltpu.VMEM((_CH, _K), jnp.int32),
        pltpu.VMEM((_CH, _K), jnp.int32),
        pltpu.VMEM((_CH, _K), jnp.float32),
        pltpu.VMEM((_K, _D), jnp.bfloat16),
        pltpu.VMEM((_K, _D), jnp.bfloat16),
        pltpu.VMEM((_K, _D), jnp.float32),
        pltpu.VMEM((_K, _D), jnp.float32),
        pltpu.VMEM_SHARED((_NP, _D), jnp.float32),
        pltpu.SemaphoreType.DMA,
        pltpu.SemaphoreType.DMA,
        pltpu.SemaphoreType.DMA,
        pltpu.SemaphoreType.DMA,
    ],
)
def _layer_kernel(h_hbm, src_hbm, dst_hbm, w_hbm, out_hbm,
                  src_v, dst_v, w_v, brows0, brows1, rows0, rows1, acc,
                  gs0, gs1, ss0, ss1):
    cid = lax.axis_index("c")
    sid = lax.axis_index("s")
    wid = cid * _NS + sid
    pltpu.sync_copy(src_hbm.at[wid], src_v)
    pltpu.sync_copy(dst_hbm.at[wid], dst_v)
    pltpu.sync_copy(w_hbm.at[wid], w_v)

    def zb(i, _):
        for j in range(_D // 16):
            rows0[i, pl.ds(j * 16, 16)] = jnp.zeros((16,), jnp.float32)
        return 0

    lax.fori_loop(0, _K, zb, 0)
    for r in range(_RPT // _K):
        pltpu.sync_copy(rows0, acc.at[pl.ds(sid * _RPT + r * _K, _K)])
    plsc.subcore_barrier()

    def scale(c, brows, rows):
        for off, ls in _WGROUPS:
            wv = w_v[c, pl.ds(off, 16)]
            for l in range(ls, 16):
                k = off + l
                s = wv[l]
                for g in range(_D // 32):
                    v = brows[k, pl.ds(g * 32, 32)]
                    a, b = plsc.unpack(v, format=plsc.PackFormat.INTERLEAVED)
                    rows[k, pl.ds(g * 32, 16)] = a * s
                    rows[k, pl.ds(g * 32 + 16, 16)] = b * s

    def maybe_gather(c, brows, sem):
        @pl.when(c < _CH)
        def _():
            pltpu.async_copy(h_hbm.at[src_v.at[c]], brows, sem)

    pltpu.async_copy(h_hbm.at[src_v.at[0]], brows0, gs0)
    pltpu.async_copy(h_hbm.at[src_v.at[1]], brows1, gs1)

    def pair(cc, _):
        c0 = cc * 2
        c1 = c0 + 1
        pltpu.make_async_copy(h_hbm.at[src_v.at[c0]], brows0, gs0).wait()
        scale(c0, brows0, rows0)
        pltpu.async_copy(rows0, acc.at[dst_v.at[c0]], ss0, add=True)
        pltpu.make_async_copy(h_hbm.at[src_v.at[c1]], brows1, gs1).wait()
        scale(c1, brows1, rows1)
        pltpu.async_copy(rows1, acc.at[dst_v.at[c1]], ss1, add=True)
        pltpu.make_async_copy(rows0, acc.at[dst_v.at[c0]], ss0).wait()
        maybe_gather(c0 + 2, brows0, gs0)
        pltpu.make_async_copy(rows1, acc.at[dst_v.at[c1]], ss1).wait()
        maybe_gather(c1 + 2, brows1, gs1)
        return 0

    lax.fori_loop(0, _CH // 2, pair, 0)
    plsc.subcore_barrier()
    pltpu.sync_copy(acc.at[pl.ds(sid * _RPT, _RPT)], out_hbm.at[cid, sid])


_PREC = jax.lax.Precision.HIGHEST


def _tc_first_body(degp_ref, x_ref, w1_ref, norm_ref, h1_ref):
    deg = degp_ref[0, 0:_N, :] + degp_ref[1, 0:_N, :]
    norm = jax.lax.rsqrt(jnp.clip(deg, 1e-12, None))
    norm_ref[...] = norm
    h = jnp.dot(x_ref[...], w1_ref[...],
                preferred_element_type=jnp.float32, precision=_PREC)
    h1_ref[...] = (h * norm[:, 0:1]).astype(jnp.bfloat16)


def _tc_mid_body(aggp_ref, norm_ref, b_ref, w_ref, h_ref):
    agg = aggp_ref[0, 0:_N] + aggp_ref[1, 0:_N]
    norm = norm_ref[...]
    out = jnp.maximum(agg * norm[:, 1:2] + b_ref[...], 0.0)
    h = jnp.dot(out, w_ref[...],
                preferred_element_type=jnp.float32, precision=_PREC)
    h_ref[...] = (h * norm[:, 0:1]).astype(jnp.bfloat16)


def _tc_final_body(aggp_ref, norm_ref, b3_ref, x_ref, wres_ref, bres_ref,
                   wop_ref, bop_ref, out_ref):
    agg = aggp_ref[0, 0:_N] + aggp_ref[1, 0:_N]
    out3 = agg * norm_ref[...][:, 1:2] + b3_ref[...]
    res = jnp.dot(x_ref[...], wres_ref[...],
                  preferred_element_type=jnp.float32, precision=_PREC)
    res = res + bres_ref[...]
    y = jnp.maximum(out3 + res, 0.0)
    out_ref[...] = jnp.dot(y, wop_ref[...],
                           preferred_element_type=jnp.float32,
                           precision=_PREC) + bop_ref[...]


_TC_PARAMS = pltpu.CompilerParams(vmem_limit_bytes=100 * 1024 * 1024)

_tc_first = pl.pallas_call(
    _tc_first_body,
    compiler_params=_TC_PARAMS,
    out_shape=(
        jax.ShapeDtypeStruct((_N, 2), jnp.float32),
        jax.ShapeDtypeStruct((_N, _D), jnp.bfloat16),
    ),
)

_tc_mid = pl.pallas_call(
    _tc_mid_body,
    compiler_params=_TC_PARAMS,
    out_shape=jax.ShapeDtypeStruct((_N, _D), jnp.bfloat16),
)

_tc_final = pl.pallas_call(
    _tc_final_body,
    compiler_params=_TC_PARAMS,
    out_shape=jax.ShapeDtypeStruct((_N, _C), jnp.float32),
)


def kernel(x, edge_index, edge_weight, W1, b1, W2, b2, W3, b3,
           Wres, bres, Wop, bop):
    src3 = edge_index[0].reshape(_NW, _CH, _K)
    dst3 = edge_index[1].reshape(_NW, _CH, _K)
    w3 = edge_weight.reshape(_NW, _CH, _K)
    colmap = jnp.asarray(_COLMAP)
    w1p = W1[:, colmap]
    w2p = W2[:, colmap]
    w3p = W3[:, colmap]

    degp = _deg_kernel(edge_index[0], edge_index[1], edge_weight)
    degp = degp.reshape(_NC, _NP, 2)

    norm, h1 = _tc_first(degp, x, w1p)
    agg1 = _layer_kernel(h1, src3, dst3, w3).reshape(_NC, _NP, _D)
    h2 = _tc_mid(agg1, norm, b1.reshape(1, _D), w2p)
    agg2 = _layer_kernel(h2, src3, dst3, w3).reshape(_NC, _NP, _D)
    h3 = _tc_mid(agg2, norm, b2.reshape(1, _D), w3p)
    agg3 = _layer_kernel(h3, src3, dst3, w3).reshape(_NC, _NP, _D)
    out = _tc_final(agg3, norm, b3.reshape(1, _D), x, Wres,
                    bres.reshape(1, _D), Wop, bop.reshape(1, _C))
    return out


# speedup vs baseline: 10.3045x; 1.1668x over previous
import functools

import jax
import jax.numpy as jnp
import numpy as np
from jax import lax
from jax.experimental import pallas as pl
from jax.experimental.pallas import tpu as pltpu
from jax.experimental.pallas import tpu_sc as plsc

_N = 10000
_E = 320000
_D = 128
_C = 64
_NC = 2
_NS = 16
_NW = _NC * _NS
_T = _E // _NW
_K = 40
_CH = _T // _K
_KD = 80
_CHD = _T // _KD
_NP = 10240
_RPT = _NP // _NS

_WGROUPS = ((0, 0), (16, 0), (24, 8))

_COLMAP = np.empty((_D,), np.int32)
for _g in range(_D // 32):
    for _i in range(16):
        _COLMAP[32 * _g + 2 * _i] = 32 * _g + _i
        _COLMAP[32 * _g + 2 * _i + 1] = 32 * _g + 16 + _i

_mesh = plsc.VectorSubcoreMesh(core_axis_name="c", subcore_axis_name="s")


_SEG = 2 * _NP // _NS


@functools.partial(
    pl.kernel,
    out_type=jax.ShapeDtypeStruct((_NC, 2 * _NP), jnp.float32),
    mesh=_mesh,
    compiler_params=pltpu.CompilerParams(use_tc_tiling_on_sc=False, needs_layout_passes=False),
    scratch_types=[
        pltpu.VMEM((_T,), jnp.int32),
        pltpu.VMEM((_T,), jnp.int32),
        pltpu.VMEM((_T,), jnp.float32),
        pltpu.VMEM((2 * _NP,), jnp.float32),
        pltpu.VMEM((_SEG,), jnp.float32),
        pltpu.VMEM_SHARED((_NS, 2 * _NP), jnp.float32),
    ],
)
def _deg_kernel(src_hbm, dst_hbm, w_hbm, out_hbm,
                src_v, dst_v, w_v, acc, seg, stage):
    cid = lax.axis_index("c")
    sid = lax.axis_index("s")
    wid = cid * _NS + sid
    base = wid * _T
    pltpu.sync_copy(src_hbm.at[pl.ds(base, _T)], src_v)
    pltpu.sync_copy(dst_hbm.at[pl.ds(base, _T)], dst_v)
    pltpu.sync_copy(w_hbm.at[pl.ds(base, _T)], w_v)

    def zero(i, _):
        acc[pl.ds(i * 16, 16)] = jnp.zeros((16,), jnp.float32)
        return 0

    lax.fori_loop(0, 2 * _NP // 16, zero, 0)

    def edge(i, _):
        sl = pl.ds(i * 16, 16)
        sv = src_v[sl]
        dv = dst_v[sl]
        wv = w_v[sl]
        plsc.addupdate_scatter(acc, [sv * 2], wv)
        plsc.addupdate_scatter(acc, [dv * 2 + 1], wv)
        return 0

    lax.fori_loop(0, _T // 16, edge, 0)

    pltpu.sync_copy(acc, stage.at[sid])
    plsc.subcore_barrier()
    off = sid * _SEG

    def peer(t, _):
        tt = lax.rem(sid + t, _NS)
        pltpu.sync_copy(stage.at[tt, pl.ds(off, _SEG)], seg)

        def add16(i, _):
            sl = pl.ds(off + i * 16, 16)
            acc[sl] = acc[sl] + seg[pl.ds(i * 16, 16)]
            return 0

        lax.fori_loop(0, _SEG // 16, add16, 0)
        return 0

    lax.fori_loop(1, _NS, peer, 0)
    pltpu.sync_copy(acc.at[pl.ds(off, _SEG)], out_hbm.at[cid, pl.ds(off, _SEG)])


@functools.partial(
    pl.kernel,
    out_type=jax.ShapeDtypeStruct((_NC, _NS, _RPT, _D), jnp.float32),
    mesh=_mesh,
    compiler_params=pltpu.CompilerParams(use_tc_tiling_on_sc=False, needs_layout_passes=False),
    scratch_types=[
        pltpu.VMEM((_CH, _K), jnp.int32),
        pltpu.VMEM((_CH, _K), jnp.int32),
        pltpu.VMEM((_CH, _K), jnp.float32),
        pltpu.VMEM((_K, _D), jnp.bfloat16),
        pltpu.VMEM((_K, _D), jnp.bfloat16),
        pltpu.VMEM((_K, _D), jnp.float32),
        pltpu.VMEM((_K, _D), jnp.float32),
        pltpu.VMEM_SHARED((_NP, _D), jnp.float32),
        pltpu.SemaphoreType.DMA,
        pltpu.SemaphoreType.DMA,
        pltpu.SemaphoreType.DMA,
        pltpu.SemaphoreType.DMA,
    ],
)
def _layer_kernel(h_hbm, src_hbm, dst_hbm, w_hbm, out_hbm,
                  src_v, dst_v, w_v, brows0, brows1, rows0, rows1, acc,
                  gs0, gs1, ss0, ss1):
    cid = lax.axis_index("c")
    sid = lax.axis_index("s")
    wid = cid * _NS + sid
    pltpu.sync_copy(src_hbm.at[wid], src_v)
    pltpu.sync_copy(dst_hbm.at[wid], dst_v)
    pltpu.sync_copy(w_hbm.at[wid], w_v)

    def zb(i, _):
        for j in range(_D // 16):
            rows0[i, pl.ds(j * 16, 16)] = jnp.zeros((16,), jnp.float32)
        return 0

    lax.fori_loop(0, _K, zb, 0)
    for r in range(_RPT // _K):
        pltpu.sync_copy(rows0, acc.at[pl.ds(sid * _RPT + r * _K, _K)])
    plsc.subcore_barrier()

    def scale(c, brows, rows):
        for off, ls in _WGROUPS:
            wv = w_v[c, pl.ds(off, 16)]
            for l in range(ls, 16):
                k = off + l
                s = wv[l]
                for g in range(_D // 32):
                    v = brows[k, pl.ds(g * 32, 32)]
                    a, b = plsc.unpack(v, format=plsc.PackFormat.INTERLEAVED)
                    rows[k, pl.ds(g * 32, 16)] = a * s
                    rows[k, pl.ds(g * 32 + 16, 16)] = b * s

    def maybe_gather(c, brows, sem):
        @pl.when(c < _CH)
        def _():
            pltpu.async_copy(h_hbm.at[src_v.at[c]], brows, sem)

    def do_chunk(c, brows, rows, gsem, ssem, first):
        pltpu.make_async_copy(h_hbm.at[src_v.at[c]], brows, gsem).wait()
        if not first:
            pltpu.make_async_copy(rows, acc.at[dst_v.at[c - 2]], ssem).wait()
        scale(c, brows, rows)
        maybe_gather(c + 2, brows, gsem)
        pltpu.async_copy(rows, acc.at[dst_v.at[c]], ssem, add=True)

    pltpu.async_copy(h_hbm.at[src_v.at[0]], brows0, gs0)
    pltpu.async_copy(h_hbm.at[src_v.at[1]], brows1, gs1)
    do_chunk(0, brows0, rows0, gs0, ss0, True)
    do_chunk(1, brows1, rows1, gs1, ss1, True)

    def pair(cc, _):
        do_chunk(cc * 2, brows0, rows0, gs0, ss0, False)
        do_chunk(cc * 2 + 1, brows1, rows1, gs1, ss1, False)
        return 0

    lax.fori_loop(1, _CH // 2, pair, 0)
    pltpu.make_async_copy(rows0, acc.at[dst_v.at[_CH - 2]], ss0).wait()
    pltpu.make_async_copy(rows1, acc.at[dst_v.at[_CH - 1]], ss1).wait()
    plsc.subcore_barrier()
    pltpu.sync_copy(acc.at[pl.ds(sid * _RPT, _RPT)], out_hbm.at[cid, sid])


_PREC = jax.lax.Precision.HIGHEST


def _tc_first_body(degp_ref, x_ref, w1_ref, norm_ref, h1_ref):
    deg = degp_ref[0, 0:_N, :] + degp_ref[1, 0:_N, :]
    norm = jax.lax.rsqrt(jnp.clip(deg, 1e-12, None))
    norm_ref[...] = norm
    h = jnp.dot(x_ref[...], w1_ref[...],
                preferred_element_type=jnp.float32, precision=_PREC)
    h1_ref[...] = (h * norm[:, 0:1]).astype(jnp.bfloat16)


def _tc_mid_body(aggp_ref, norm_ref, b_ref, w_ref, h_ref):
    agg = aggp_ref[0, 0:_N] + aggp_ref[1, 0:_N]
    norm = norm_ref[...]
    out = jnp.maximum(agg * norm[:, 1:2] + b_ref[...], 0.0)
    h = jnp.dot(out, w_ref[...],
                preferred_element_type=jnp.float32, precision=_PREC)
    h_ref[...] = (h * norm[:, 0:1]).astype(jnp.bfloat16)


def _tc_final_body(aggp_ref, norm_ref, b3_ref, x_ref, wres_ref, bres_ref,
                   wop_ref, bop_ref, out_ref):
    agg = aggp_ref[0, 0:_N] + aggp_ref[1, 0:_N]
    out3 = agg * norm_ref[...][:, 1:2] + b3_ref[...]
    res = jnp.dot(x_ref[...], wres_ref[...],
                  preferred_element_type=jnp.float32, precision=_PREC)
    res = res + bres_ref[...]
    y = jnp.maximum(out3 + res, 0.0)
    out_ref[...] = jnp.dot(y, wop_ref[...],
                           preferred_element_type=jnp.float32,
                           precision=_PREC) + bop_ref[...]


_TC_PARAMS = pltpu.CompilerParams(vmem_limit_bytes=100 * 1024 * 1024)

_tc_first = pl.pallas_call(
    _tc_first_body,
    compiler_params=_TC_PARAMS,
    out_shape=(
        jax.ShapeDtypeStruct((_N, 2), jnp.float32),
        jax.ShapeDtypeStruct((_N, _D), jnp.bfloat16),
    ),
)

_tc_mid = pl.pallas_call(
    _tc_mid_body,
    compiler_params=_TC_PARAMS,
    out_shape=jax.ShapeDtypeStruct((_N, _D), jnp.bfloat16),
)

_tc_final = pl.pallas_call(
    _tc_final_body,
    compiler_params=_TC_PARAMS,
    out_shape=jax.ShapeDtypeStruct((_N, _C), jnp.float32),
)


def kernel(x, edge_index, edge_weight, W1, b1, W2, b2, W3, b3,
           Wres, bres, Wop, bop):
    src3 = edge_index[0].reshape(_NW, _CH, _K)
    dst3 = edge_index[1].reshape(_NW, _CH, _K)
    w3 = edge_weight.reshape(_NW, _CH, _K)
    colmap = jnp.asarray(_COLMAP)
    w1p = W1[:, colmap]
    w2p = W2[:, colmap]
    w3p = W3[:, colmap]

    degp = _deg_kernel(edge_index[0], edge_index[1], edge_weight)
    degp = degp.reshape(_NC, _NP, 2)

    norm, h1 = _tc_first(degp, x, w1p)
    agg1 = _layer_kernel(h1, src3, dst3, w3).reshape(_NC, _NP, _D)
    h2 = _tc_mid(agg1, norm, b1.reshape(1, _D), w2p)
    agg2 = _layer_kernel(h2, src3, dst3, w3).reshape(_NC, _NP, _D)
    h3 = _tc_mid(agg2, norm, b2.reshape(1, _D), w3p)
    agg3 = _layer_kernel(h3, src3, dst3, w3).reshape(_NC, _NP, _D)
    out = _tc_final(agg3, norm, b3.reshape(1, _D), x, Wres,
                    bres.reshape(1, _D), Wop, bop.reshape(1, _C))
    return out
